# Initial kernel scaffold; baseline (speedup 1.0000x reference)
import jax
import jax.numpy as jnp
from jax.experimental import pallas as pl


def kernel(fea, res, queues):
    raise NotImplementedError("write your pallas kernel here")


# speedup vs baseline: 1.4333x; 1.4333x over previous
import functools

import jax
import jax.numpy as jnp
from jax import lax
from jax.experimental import pallas as pl
from jax.experimental.pallas import tpu as pltpu
from jax.experimental.pallas import tpu_sc as plsc

INNER = 256
NCLS = 19
QLEN = 2975
BSZ = 8
HW = 128 * 128
NPX = BSZ * HW
TEMP = 0.2

NC, NS, L = 2, 16, 16
NW = NC * NS
PXT = NPX // NW
CHG = INNER // 4
ACC = NCLS * CHG * L
BLK = 128
NBLK = (QLEN + BLK - 1) // BLK


def _sc_mesh():
    return plsc.VectorSubcoreMesh(
        core_axis_name="c", subcore_axis_name="s", num_cores=NC, num_subcores=NS
    )


def _argmax_body(res_hbm, pred16_hbm, counts_hbm, resbuf, predbuf, cntbuf):
    wid = lax.axis_index("s") * NC + lax.axis_index("c")
    b = wid // 4
    q = wid % 4
    base = b * (NCLS * HW) + q * PXT
    for c in range(NCLS):
        pltpu.sync_copy(
            res_hbm.at[pl.ds(base + c * HW, PXT)],
            resbuf.at[pl.ds(c * PXT, PXT)],
        )

    zero = jnp.zeros((L,), jnp.float32)
    for i in range(NCLS):
        cntbuf[pl.ds(i * L, L)] = zero

    lane = lax.iota(jnp.int32, L)
    ones = jnp.ones((L,), jnp.float32)

    @pl.loop(0, PXT // L)
    def _px(v):
        off = v * L
        best = resbuf[pl.ds(off, L)]
        bidx = jnp.zeros((L,), jnp.int32)
        for c in range(1, NCLS):
            x = resbuf[pl.ds(c * PXT + off, L)]
            gt = x > best
            best = jnp.where(gt, x, best)
            bidx = jnp.where(gt, jnp.full((L,), c, jnp.int32), bidx)
        predbuf[pl.ds(off, L)] = bidx * (CHG * L) + lane
        plsc.addupdate_scatter(cntbuf, [bidx * L + lane], ones)

    pltpu.sync_copy(predbuf, pred16_hbm.at[pl.ds(wid * PXT, PXT)])
    pltpu.sync_copy(cntbuf, counts_hbm.at[wid])


@functools.cache
def _argmax_call():
    return pl.kernel(
        _argmax_body,
        out_type=[
            jax.ShapeDtypeStruct((NPX,), jnp.int32),
            jax.ShapeDtypeStruct((NW, NCLS * L), jnp.float32),
        ],
        mesh=_sc_mesh(),
        scratch_types=[
            pltpu.VMEM((NCLS * PXT,), jnp.float32),
            pltpu.VMEM((PXT,), jnp.int32),
            pltpu.VMEM((NCLS * L,), jnp.float32),
        ],
        compiler_params=pltpu.CompilerParams(needs_layout_passes=False),
    )


def _segsum_body(fea_hbm, pred16_hbm, part_hbm, pbuf, fbuf, acc):
    wid = lax.axis_index("s") * NC + lax.axis_index("c")
    b = wid // 4
    cg = wid % 4

    pltpu.sync_copy(pred16_hbm.at[pl.ds(b * HW, HW)], pbuf)

    zero = jnp.zeros((L,), jnp.float32)

    @pl.loop(0, ACC // L)
    def _z(i):
        acc[pl.ds(i * L, L)] = zero

    febase = (b * INNER + cg * CHG) * HW

    @pl.loop(0, CHG)
    def _ch(k):
        pltpu.sync_copy(fea_hbm.at[pl.ds(febase + k * HW, HW)], fbuf)
        koff = k * L

        @pl.loop(0, HW // L, unroll=8)
        def _px(v):
            off = v * L
            idx = pbuf[pl.ds(off, L)] + koff
            plsc.addupdate_scatter(acc, [idx], fbuf[pl.ds(off, L)])

    pltpu.sync_copy(acc, part_hbm.at[wid])


@functools.cache
def _segsum_call():
    return pl.kernel(
        _segsum_body,
        out_type=jax.ShapeDtypeStruct((NW, ACC), jnp.float32),
        mesh=_sc_mesh(),
        scratch_types=[
            pltpu.VMEM((HW,), jnp.int32),
            pltpu.VMEM((HW,), jnp.float32),
            pltpu.VMEM((ACC,), jnp.float32),
        ],
        compiler_params=pltpu.CompilerParams(needs_layout_passes=False),
    )


def _loss_body(pref, cref, qref, out, keys_s, l0s, sacc):
    j = pl.program_id(0)
    invt = jnp.float32(1.0 / TEMP)

    @pl.when(j == 0)
    def _init():
        sums = jnp.sum(pref[...], axis=2)
        counts = jnp.sum(cref[...], axis=1)
        safe = jnp.where(counts > 0, counts, jnp.ones_like(counts))
        k0 = sums / safe[:, None]
        nrm = jnp.sqrt(jnp.sum(k0 * k0, axis=1, keepdims=True))
        kk = k0 / jnp.maximum(nrm, 1e-12)
        keys_s[...] = kk
        l0s[...] = kk * qref[:, :, 0] * invt
        sacc[...] = jnp.zeros((NCLS, INNER), jnp.float32)
        out[0, 0] = jnp.float32(0.0)

    qb = qref[...]
    qsum = jnp.sum(qb, axis=0)
    kk = keys_s[...]
    col = lax.broadcasted_iota(jnp.int32, (NCLS, INNER, BLK), 2) + j * BLK
    x1 = kk[:, :, None] * qb * invt
    x2 = kk[:, :, None] * (qsum[None, :, :] - qb) * invt
    e = jnp.where(col < QLEN, jnp.exp(x1) + jnp.exp(x2), jnp.float32(0.0))
    sacc[...] = sacc[...] + jnp.sum(e, axis=2)

    @pl.when(j == NBLK - 1)
    def _fin():
        counts = jnp.sum(cref[...], axis=1)
        pres = (counts > 0).astype(jnp.float32)
        loss = jnp.sum(pres[:, None] * (jnp.log(sacc[...]) - l0s[...]))
        out[0, 0] = loss / jnp.float32(INNER)


def _loss_call(p6, c3, queues, interpret=False):
    return pl.pallas_call(
        _loss_body,
        grid=(NBLK,),
        in_specs=[
            pl.BlockSpec((NCLS, INNER, BSZ * L), lambda j: (0, 0, 0)),
            pl.BlockSpec((NCLS, NW * L), lambda j: (0, 0)),
            pl.BlockSpec((NCLS, INNER, BLK), lambda j: (0, 0, j)),
        ],
        out_specs=pl.BlockSpec(memory_space=pltpu.SMEM),
        out_shape=jax.ShapeDtypeStruct((1, 1), jnp.float32),
        scratch_shapes=[
            pltpu.VMEM((NCLS, INNER), jnp.float32),
            pltpu.VMEM((NCLS, INNER), jnp.float32),
            pltpu.VMEM((NCLS, INNER), jnp.float32),
        ],
        compiler_params=pltpu.CompilerParams(
            dimension_semantics=("arbitrary",)
        ),
        interpret=interpret,
    )(p6, c3, queues)


def kernel(fea, res, queues):
    res_flat = res.reshape(-1)
    fea_flat = fea.reshape(-1)
    pred16, countsp = _argmax_call()(res_flat)
    partials = _segsum_call()(fea_flat, pred16)
    p6 = (
        partials.reshape(BSZ, 4, NCLS, CHG, L)
        .transpose(2, 1, 3, 0, 4)
        .reshape(NCLS, INNER, BSZ * L)
    )
    c3 = countsp.reshape(NW, NCLS, L).transpose(1, 0, 2).reshape(NCLS, NW * L)
    loss = _loss_call(p6, c3, queues)[0, 0]
    return res, loss


# speedup vs baseline: 3.6330x; 2.5347x over previous
import functools

import jax
import jax.numpy as jnp
from jax import lax
from jax.experimental import pallas as pl
from jax.experimental.pallas import tpu as pltpu
from jax.experimental.pallas import tpu_sc as plsc

INNER = 256
NCLS = 19
QLEN = 2975
BSZ = 8
HW = 128 * 128
NPX = BSZ * HW
TEMP = 0.2

NC, NS, L = 2, 16, 16
NW = NC * NS
PXT = NPX // NW
CHG = INNER // 4
ACC = NCLS * CHG * L
BLK = 128
NBLK = (QLEN + BLK - 1) // BLK


def _sc_mesh():
    return plsc.VectorSubcoreMesh(
        core_axis_name="c", subcore_axis_name="s", num_cores=NC, num_subcores=NS
    )


def _argmax_body(res_hbm, pred16_hbm, counts_hbm, resbuf, predbuf, cntbuf):
    wid = lax.axis_index("s") * NC + lax.axis_index("c")
    b = wid // 4
    q = wid % 4
    base = b * (NCLS * HW) + q * PXT
    for c in range(NCLS):
        pltpu.sync_copy(
            res_hbm.at[pl.ds(base + c * HW, PXT)],
            resbuf.at[pl.ds(c * PXT, PXT)],
        )

    zero = jnp.zeros((L,), jnp.float32)
    for i in range(NCLS):
        cntbuf[pl.ds(i * L, L)] = zero

    lane = lax.iota(jnp.int32, L)
    ones = jnp.ones((L,), jnp.float32)

    @pl.loop(0, PXT // L)
    def _px(v):
        off = v * L
        best = resbuf[pl.ds(off, L)]
        bidx = jnp.zeros((L,), jnp.int32)
        for c in range(1, NCLS):
            x = resbuf[pl.ds(c * PXT + off, L)]
            gt = x > best
            best = jnp.where(gt, x, best)
            bidx = jnp.where(gt, jnp.full((L,), c, jnp.int32), bidx)
        predbuf[pl.ds(off, L)] = bidx * (CHG * L) + lane
        plsc.addupdate_scatter(cntbuf, [bidx * L + lane], ones)

    pltpu.sync_copy(predbuf, pred16_hbm.at[pl.ds(wid * PXT, PXT)])
    pltpu.sync_copy(cntbuf, counts_hbm.at[wid])


@functools.cache
def _argmax_call():
    return pl.kernel(
        _argmax_body,
        out_type=[
            jax.ShapeDtypeStruct((NPX,), jnp.int32),
            jax.ShapeDtypeStruct((NW, NCLS * L), jnp.float32),
        ],
        mesh=_sc_mesh(),
        scratch_types=[
            pltpu.VMEM((NCLS * PXT,), jnp.float32),
            pltpu.VMEM((PXT,), jnp.int32),
            pltpu.VMEM((NCLS * L,), jnp.float32),
        ],
        compiler_params=pltpu.CompilerParams(needs_layout_passes=False),
    )


NPAIR = CHG // 2


def _segsum_body(fea_hbm, pred16_hbm, part_hbm, pbuf, fbuf, acc, sem0, sem1):
    wid = lax.axis_index("s") * NC + lax.axis_index("c")
    b = wid // 4
    cg = wid % 4

    pltpu.sync_copy(pred16_hbm.at[pl.ds(b * HW, HW)], pbuf)

    zero = jnp.zeros((L,), jnp.float32)

    @pl.loop(0, ACC // L)
    def _z(i):
        acc[pl.ds(i * L, L)] = zero

    febase = (b * INNER + cg * CHG) * HW
    sems = (sem0, sem1)

    def start_pair(p, slot):
        base = febase + p * (2 * HW)
        pltpu.async_copy(
            fea_hbm.at[pl.ds(base, HW)],
            fbuf.at[pl.ds(slot * 2 * HW, HW)],
            sems[slot],
        )
        pltpu.async_copy(
            fea_hbm.at[pl.ds(base + HW, HW)],
            fbuf.at[pl.ds(slot * 2 * HW + HW, HW)],
            sems[slot],
        )

    def wait_pair(slot):
        for j in range(2):
            pltpu.make_async_copy(
                fea_hbm.at[pl.ds(0, HW)],
                fbuf.at[pl.ds(slot * 2 * HW + j * HW, HW)],
                sems[slot],
            ).wait()

    start_pair(0, 0)

    @pl.loop(0, NPAIR, step=2)
    def _pair(p0):
        for r in range(2):
            p = p0 + r

            @pl.when(p + 1 < NPAIR)
            def _():
                start_pair(p + 1, 1 - r)

            wait_pair(r)
            koff = p * (2 * L)
            fb = r * 2 * HW

            @plsc.parallel_loop(0, HW // L, unroll=8)
            def _px(v):
                off = v * L
                idx = pbuf[pl.ds(off, L)] + koff
                plsc.addupdate_scatter(acc, [idx], fbuf[pl.ds(fb + off, L)])
                plsc.addupdate_scatter(
                    acc, [idx + L], fbuf[pl.ds(fb + HW + off, L)]
                )

    pltpu.sync_copy(acc, part_hbm.at[wid])


@functools.cache
def _segsum_call():
    return pl.kernel(
        _segsum_body,
        out_type=jax.ShapeDtypeStruct((NW, ACC), jnp.float32),
        mesh=_sc_mesh(),
        scratch_types=[
            pltpu.VMEM((HW,), jnp.int32),
            pltpu.VMEM((4 * HW,), jnp.float32),
            pltpu.VMEM((ACC,), jnp.float32),
            pltpu.SemaphoreType.DMA,
            pltpu.SemaphoreType.DMA,
        ],
        compiler_params=pltpu.CompilerParams(needs_layout_passes=False),
    )


def _loss_body(pref, cref, qref, out, keys_s, l0s, sacc):
    j = pl.program_id(0)
    invt = jnp.float32(1.0 / TEMP)

    @pl.when(j == 0)
    def _init():
        sums = jnp.sum(pref[...], axis=2)
        counts = jnp.sum(cref[...], axis=1)
        safe = jnp.where(counts > 0, counts, jnp.ones_like(counts))
        k0 = sums / safe[:, None]
        nrm = jnp.sqrt(jnp.sum(k0 * k0, axis=1, keepdims=True))
        kk = k0 / jnp.maximum(nrm, 1e-12)
        keys_s[...] = kk
        l0s[...] = kk * qref[:, :, 0] * invt
        sacc[...] = jnp.zeros((NCLS, INNER), jnp.float32)
        out[0, 0] = jnp.float32(0.0)

    qb = qref[...]
    qsum = jnp.sum(qb, axis=0)
    kk = keys_s[...]
    col = lax.broadcasted_iota(jnp.int32, (NCLS, INNER, BLK), 2) + j * BLK
    x1 = kk[:, :, None] * qb * invt
    x2 = kk[:, :, None] * (qsum[None, :, :] - qb) * invt
    e = jnp.where(col < QLEN, jnp.exp(x1) + jnp.exp(x2), jnp.float32(0.0))
    sacc[...] = sacc[...] + jnp.sum(e, axis=2)

    @pl.when(j == NBLK - 1)
    def _fin():
        counts = jnp.sum(cref[...], axis=1)
        pres = (counts > 0).astype(jnp.float32)
        loss = jnp.sum(pres[:, None] * (jnp.log(sacc[...]) - l0s[...]))
        out[0, 0] = loss / jnp.float32(INNER)


def _loss_call(p6, c3, queues, interpret=False):
    return pl.pallas_call(
        _loss_body,
        grid=(NBLK,),
        in_specs=[
            pl.BlockSpec((NCLS, INNER, BSZ * L), lambda j: (0, 0, 0)),
            pl.BlockSpec((NCLS, NW * L), lambda j: (0, 0)),
            pl.BlockSpec((NCLS, INNER, BLK), lambda j: (0, 0, j)),
        ],
        out_specs=pl.BlockSpec(memory_space=pltpu.SMEM),
        out_shape=jax.ShapeDtypeStruct((1, 1), jnp.float32),
        scratch_shapes=[
            pltpu.VMEM((NCLS, INNER), jnp.float32),
            pltpu.VMEM((NCLS, INNER), jnp.float32),
            pltpu.VMEM((NCLS, INNER), jnp.float32),
        ],
        compiler_params=pltpu.CompilerParams(
            dimension_semantics=("arbitrary",)
        ),
        interpret=interpret,
    )(p6, c3, queues)


def kernel(fea, res, queues):
    res_flat = res.reshape(-1)
    fea_flat = fea.reshape(-1)
    pred16, countsp = _argmax_call()(res_flat)
    partials = _segsum_call()(fea_flat, pred16)
    p6 = (
        partials.reshape(BSZ, 4, NCLS, CHG, L)
        .transpose(2, 1, 3, 0, 4)
        .reshape(NCLS, INNER, BSZ * L)
    )
    c3 = countsp.reshape(NW, NCLS, L).transpose(1, 0, 2).reshape(NCLS, NW * L)
    loss = _loss_call(p6, c3, queues)[0, 0]
    return res, loss


# speedup vs baseline: 3.6908x; 1.0159x over previous
import functools

import jax
import jax.numpy as jnp
from jax import lax
from jax.experimental import pallas as pl
from jax.experimental.pallas import tpu as pltpu
from jax.experimental.pallas import tpu_sc as plsc

INNER = 256
NCLS = 19
QLEN = 2975
BSZ = 8
HW = 128 * 128
NPX = BSZ * HW
TEMP = 0.2

NC, NS, L = 2, 16, 16
NW = NC * NS
PXT = NPX // NW
CHG = INNER // 4
ACC = NCLS * CHG * L
BLK = 128
NBLK = (QLEN + BLK - 1) // BLK


def _sc_mesh():
    return plsc.VectorSubcoreMesh(
        core_axis_name="c", subcore_axis_name="s", num_cores=NC, num_subcores=NS
    )


def _argmax_body(res_hbm, pred16_hbm, counts_hbm, resbuf, predbuf, cntbuf):
    wid = lax.axis_index("s") * NC + lax.axis_index("c")
    b = wid // 4
    q = wid % 4
    base = b * (NCLS * HW) + q * PXT
    for c in range(NCLS):
        pltpu.sync_copy(
            res_hbm.at[pl.ds(base + c * HW, PXT)],
            resbuf.at[pl.ds(c * PXT, PXT)],
        )

    zero = jnp.zeros((L,), jnp.float32)
    for i in range(NCLS):
        cntbuf[pl.ds(i * L, L)] = zero

    lane = lax.iota(jnp.int32, L)
    ones = jnp.ones((L,), jnp.float32)

    @pl.loop(0, PXT // L)
    def _px(v):
        off = v * L
        best = resbuf[pl.ds(off, L)]
        bidx = jnp.zeros((L,), jnp.int32)
        for c in range(1, NCLS):
            x = resbuf[pl.ds(c * PXT + off, L)]
            gt = x > best
            best = jnp.where(gt, x, best)
            bidx = jnp.where(gt, jnp.full((L,), c, jnp.int32), bidx)
        predbuf[pl.ds(off, L)] = bidx * (CHG * L) + lane
        plsc.addupdate_scatter(cntbuf, [bidx * L + lane], ones)

    pltpu.sync_copy(predbuf, pred16_hbm.at[pl.ds(wid * PXT, PXT)])
    pltpu.sync_copy(cntbuf, counts_hbm.at[wid])


@functools.cache
def _argmax_call():
    return pl.kernel(
        _argmax_body,
        out_type=[
            jax.ShapeDtypeStruct((NPX,), jnp.int32),
            jax.ShapeDtypeStruct((NW, NCLS * L), jnp.float32),
        ],
        mesh=_sc_mesh(),
        scratch_types=[
            pltpu.VMEM((NCLS * PXT,), jnp.float32),
            pltpu.VMEM((PXT,), jnp.int32),
            pltpu.VMEM((NCLS * L,), jnp.float32),
        ],
        compiler_params=pltpu.CompilerParams(needs_layout_passes=False),
    )


NPAIR = CHG // 2


def _segsum_body(fea_hbm, pred16_hbm, part_hbm, pbuf, fbuf, acc, sem0, sem1):
    wid = lax.axis_index("s") * NC + lax.axis_index("c")
    b = wid // 4
    cg = wid % 4

    pltpu.sync_copy(pred16_hbm.at[pl.ds(b * HW, HW)], pbuf)

    zero = jnp.zeros((L,), jnp.float32)

    @pl.loop(0, ACC // L)
    def _z(i):
        acc[pl.ds(i * L, L)] = zero

    febase = (b * INNER + cg * CHG) * HW
    sems = (sem0, sem1)

    def start_pair(p, slot):
        base = febase + p * (2 * HW)
        pltpu.async_copy(
            fea_hbm.at[pl.ds(base, HW)],
            fbuf.at[pl.ds(slot * 2 * HW, HW)],
            sems[slot],
        )
        pltpu.async_copy(
            fea_hbm.at[pl.ds(base + HW, HW)],
            fbuf.at[pl.ds(slot * 2 * HW + HW, HW)],
            sems[slot],
        )

    def wait_pair(slot):
        for j in range(2):
            pltpu.make_async_copy(
                fea_hbm.at[pl.ds(0, HW)],
                fbuf.at[pl.ds(slot * 2 * HW + j * HW, HW)],
                sems[slot],
            ).wait()

    start_pair(0, 0)

    @pl.loop(0, NPAIR, step=2)
    def _pair(p0):
        for r in range(2):
            p = p0 + r

            @pl.when(p + 1 < NPAIR)
            def _():
                start_pair(p + 1, 1 - r)

            wait_pair(r)
            koff = p * (2 * L)
            fb = r * 2 * HW

            @plsc.parallel_loop(0, HW // L, unroll=8)
            def _px(v):
                off = v * L
                idx = pbuf[pl.ds(off, L)] + koff
                plsc.addupdate_scatter(acc, [idx], fbuf[pl.ds(fb + off, L)])
                plsc.addupdate_scatter(
                    acc, [idx + L], fbuf[pl.ds(fb + HW + off, L)]
                )

    pltpu.sync_copy(acc, part_hbm.at[wid])


@functools.cache
def _segsum_call():
    return pl.kernel(
        _segsum_body,
        out_type=jax.ShapeDtypeStruct((NW, ACC), jnp.float32),
        mesh=_sc_mesh(),
        scratch_types=[
            pltpu.VMEM((HW,), jnp.int32),
            pltpu.VMEM((4 * HW,), jnp.float32),
            pltpu.VMEM((ACC,), jnp.float32),
            pltpu.SemaphoreType.DMA,
            pltpu.SemaphoreType.DMA,
        ],
        compiler_params=pltpu.CompilerParams(needs_layout_passes=False),
    )


def _loss_body(pref, cref, qref, out, keys_s, l0s, sacc3):
    j = pl.program_id(0)
    invt = jnp.float32(1.0 / TEMP)

    @pl.when(j == 0)
    def _init():
        sums = jnp.sum(pref[...], axis=2)
        counts = jnp.sum(cref[...], axis=1)
        safe = jnp.where(counts > 0, counts, jnp.ones_like(counts))
        k0 = sums / safe[:, None]
        nrm = jnp.sqrt(jnp.sum(k0 * k0, axis=1, keepdims=True))
        ks = k0 / jnp.maximum(nrm, 1e-12) * invt
        keys_s[...] = ks
        l0s[...] = ks * qref[:, :, 0]
        out[0, 0] = jnp.float32(0.0)

    qb = qref[...]
    qsum = jnp.sum(qb, axis=0)
    ks = keys_s[...]
    x1 = ks[:, :, None] * qb
    x2 = ks[:, :, None] * qsum[None, :, :] - x1
    e = jnp.exp(x1) + jnp.exp(x2)

    @pl.when(j == 0)
    def _acc0():
        sacc3[...] = e

    @pl.when(jnp.logical_and(j > 0, j < NBLK - 1))
    def _accmid():
        sacc3[...] = sacc3[...] + e

    @pl.when(j == NBLK - 1)
    def _fin():
        col = lax.broadcasted_iota(jnp.int32, (NCLS, INNER, BLK), 2) + j * BLK
        em = jnp.where(col < QLEN, e, jnp.float32(0.0))
        s2 = jnp.sum(sacc3[...] + em, axis=2)
        counts = jnp.sum(cref[...], axis=1)
        pres = (counts > 0).astype(jnp.float32)
        loss = jnp.sum(pres[:, None] * (jnp.log(s2) - l0s[...]))
        out[0, 0] = loss / jnp.float32(INNER)


def _loss_call(p6, c3, queues, interpret=False):
    return pl.pallas_call(
        _loss_body,
        grid=(NBLK,),
        in_specs=[
            pl.BlockSpec((NCLS, INNER, BSZ * L), lambda j: (0, 0, 0)),
            pl.BlockSpec((NCLS, NW * L), lambda j: (0, 0)),
            pl.BlockSpec((NCLS, INNER, BLK), lambda j: (0, 0, j)),
        ],
        out_specs=pl.BlockSpec(memory_space=pltpu.SMEM),
        out_shape=jax.ShapeDtypeStruct((1, 1), jnp.float32),
        scratch_shapes=[
            pltpu.VMEM((NCLS, INNER), jnp.float32),
            pltpu.VMEM((NCLS, INNER), jnp.float32),
            pltpu.VMEM((NCLS, INNER, BLK), jnp.float32),
        ],
        compiler_params=pltpu.CompilerParams(
            dimension_semantics=("arbitrary",)
        ),
        interpret=interpret,
    )(p6, c3, queues)


def kernel(fea, res, queues):
    res_flat = res.reshape(-1)
    fea_flat = fea.reshape(-1)
    pred16, countsp = _argmax_call()(res_flat)
    partials = _segsum_call()(fea_flat, pred16)
    p6 = (
        partials.reshape(BSZ, 4, NCLS, CHG, L)
        .transpose(2, 1, 3, 0, 4)
        .reshape(NCLS, INNER, BSZ * L)
    )
    c3 = countsp.reshape(NW, NCLS, L).transpose(1, 0, 2).reshape(NCLS, NW * L)
    loss = _loss_call(p6, c3, queues)[0, 0]
    return res, loss


# speedup vs baseline: 3.8642x; 1.0470x over previous
import functools

import jax
import jax.numpy as jnp
from jax import lax
from jax.experimental import pallas as pl
from jax.experimental.pallas import tpu as pltpu
from jax.experimental.pallas import tpu_sc as plsc

INNER = 256
NCLS = 19
QLEN = 2975
BSZ = 8
HW = 128 * 128
NPX = BSZ * HW
TEMP = 0.2

NC, NS, L = 2, 16, 16
NW = NC * NS
PXT = NPX // NW
CHG = INNER // 4
ACC = NCLS * CHG * L
BLK = 128
NBLK = (QLEN + BLK - 1) // BLK


def _sc_mesh():
    return plsc.VectorSubcoreMesh(
        core_axis_name="c", subcore_axis_name="s", num_cores=NC, num_subcores=NS
    )


def _argmax_body(res_hbm, pred16_hbm, counts_hbm, resbuf, predbuf, cntbuf, sem):
    wid = lax.axis_index("s") * NC + lax.axis_index("c")
    b = wid // 4
    q = wid % 4
    base = b * (NCLS * HW) + q * PXT
    for c in range(NCLS):
        pltpu.async_copy(
            res_hbm.at[pl.ds(base + c * HW, PXT)],
            resbuf.at[pl.ds(c * PXT, PXT)],
            sem,
        )
    for c in range(NCLS):
        pltpu.make_async_copy(
            res_hbm.at[pl.ds(base + c * HW, PXT)],
            resbuf.at[pl.ds(c * PXT, PXT)],
            sem,
        ).wait()

    zero = jnp.zeros((L,), jnp.float32)
    for i in range(NCLS):
        cntbuf[pl.ds(i * L, L)] = zero

    lane = lax.iota(jnp.int32, L)
    ones = jnp.ones((L,), jnp.float32)

    @pl.loop(0, PXT // L)
    def _px(v):
        off = v * L
        best = resbuf[pl.ds(off, L)]
        bidx = jnp.zeros((L,), jnp.int32)
        for c in range(1, NCLS):
            x = resbuf[pl.ds(c * PXT + off, L)]
            gt = x > best
            best = jnp.where(gt, x, best)
            bidx = jnp.where(gt, jnp.full((L,), c, jnp.int32), bidx)
        predbuf[pl.ds(off, L)] = bidx * (CHG * L) + lane
        plsc.addupdate_scatter(cntbuf, [bidx * L + lane], ones)

    pltpu.sync_copy(predbuf, pred16_hbm.at[pl.ds(wid * PXT, PXT)])
    pltpu.sync_copy(cntbuf, counts_hbm.at[wid])


@functools.cache
def _argmax_call():
    return pl.kernel(
        _argmax_body,
        out_type=[
            jax.ShapeDtypeStruct((NPX,), jnp.int32),
            jax.ShapeDtypeStruct((NW, NCLS * L), jnp.float32),
        ],
        mesh=_sc_mesh(),
        scratch_types=[
            pltpu.VMEM((NCLS * PXT,), jnp.float32),
            pltpu.VMEM((PXT,), jnp.int32),
            pltpu.VMEM((NCLS * L,), jnp.float32),
            pltpu.SemaphoreType.DMA,
        ],
        compiler_params=pltpu.CompilerParams(needs_layout_passes=False),
    )


NPAIR = CHG // 2


def _segsum_body(fea_hbm, pred16_hbm, part_hbm, pbuf, fbuf, acc, sem0, sem1):
    wid = lax.axis_index("s") * NC + lax.axis_index("c")
    b = wid // 4
    cg = wid % 4

    pltpu.sync_copy(pred16_hbm.at[pl.ds(b * HW, HW)], pbuf)

    zero = jnp.zeros((L,), jnp.float32)

    @pl.loop(0, ACC // L)
    def _z(i):
        acc[pl.ds(i * L, L)] = zero

    febase = (b * INNER + cg * CHG) * HW
    sems = (sem0, sem1)

    def start_pair(p, slot):
        base = febase + p * (2 * HW)
        pltpu.async_copy(
            fea_hbm.at[pl.ds(base, HW)],
            fbuf.at[pl.ds(slot * 2 * HW, HW)],
            sems[slot],
        )
        pltpu.async_copy(
            fea_hbm.at[pl.ds(base + HW, HW)],
            fbuf.at[pl.ds(slot * 2 * HW + HW, HW)],
            sems[slot],
        )

    def wait_pair(slot):
        for j in range(2):
            pltpu.make_async_copy(
                fea_hbm.at[pl.ds(0, HW)],
                fbuf.at[pl.ds(slot * 2 * HW + j * HW, HW)],
                sems[slot],
            ).wait()

    start_pair(0, 0)

    @pl.loop(0, NPAIR, step=2)
    def _pair(p0):
        for r in range(2):
            p = p0 + r

            @pl.when(p + 1 < NPAIR)
            def _():
                start_pair(p + 1, 1 - r)

            wait_pair(r)
            koff = p * (2 * L)
            fb = r * 2 * HW

            @plsc.parallel_loop(0, HW // L, unroll=8)
            def _px(v):
                off = v * L
                idx = pbuf[pl.ds(off, L)] + koff
                plsc.addupdate_scatter(acc, [idx], fbuf[pl.ds(fb + off, L)])
                plsc.addupdate_scatter(
                    acc, [idx + L], fbuf[pl.ds(fb + HW + off, L)]
                )

    pltpu.sync_copy(acc, part_hbm.at[wid])


@functools.cache
def _segsum_call():
    return pl.kernel(
        _segsum_body,
        out_type=jax.ShapeDtypeStruct((NW, ACC), jnp.float32),
        mesh=_sc_mesh(),
        scratch_types=[
            pltpu.VMEM((HW,), jnp.int32),
            pltpu.VMEM((4 * HW,), jnp.float32),
            pltpu.VMEM((ACC,), jnp.float32),
            pltpu.SemaphoreType.DMA,
            pltpu.SemaphoreType.DMA,
        ],
        compiler_params=pltpu.CompilerParams(needs_layout_passes=False),
    )


def _loss_body(pref, cref, qref, out, keys_s, l0s, sacc3):
    j = pl.program_id(0)
    invt = jnp.float32(1.0 / TEMP)

    @pl.when(j == 0)
    def _init():
        sums = jnp.sum(pref[...], axis=2)
        counts = jnp.sum(cref[...], axis=1)
        safe = jnp.where(counts > 0, counts, jnp.ones_like(counts))
        k0 = sums / safe[:, None]
        nrm = jnp.sqrt(jnp.sum(k0 * k0, axis=1, keepdims=True))
        ks = k0 / jnp.maximum(nrm, 1e-12) * invt
        keys_s[...] = ks
        l0s[...] = ks * qref[:, :, 0]
        out[0, 0] = jnp.float32(0.0)

    qb = qref[...]
    qsum = jnp.sum(qb, axis=0)
    ks = keys_s[...]
    x1 = ks[:, :, None] * qb
    x2 = ks[:, :, None] * qsum[None, :, :] - x1
    e = jnp.exp(x1) + jnp.exp(x2)

    @pl.when(j == 0)
    def _acc0():
        sacc3[...] = e

    @pl.when(jnp.logical_and(j > 0, j < NBLK - 1))
    def _accmid():
        sacc3[...] = sacc3[...] + e

    @pl.when(j == NBLK - 1)
    def _fin():
        col = lax.broadcasted_iota(jnp.int32, (NCLS, INNER, BLK), 2) + j * BLK
        em = jnp.where(col < QLEN, e, jnp.float32(0.0))
        s2 = jnp.sum(sacc3[...] + em, axis=2)
        counts = jnp.sum(cref[...], axis=1)
        pres = (counts > 0).astype(jnp.float32)
        loss = jnp.sum(pres[:, None] * (jnp.log(s2) - l0s[...]))
        out[0, 0] = loss / jnp.float32(INNER)


def _loss_call(p6, c3, queues, interpret=False):
    return pl.pallas_call(
        _loss_body,
        grid=(NBLK,),
        in_specs=[
            pl.BlockSpec((NCLS, INNER, BSZ * L), lambda j: (0, 0, 0)),
            pl.BlockSpec((NCLS, NW * L), lambda j: (0, 0)),
            pl.BlockSpec((NCLS, INNER, BLK), lambda j: (0, 0, j)),
        ],
        out_specs=pl.BlockSpec(memory_space=pltpu.SMEM),
        out_shape=jax.ShapeDtypeStruct((1, 1), jnp.float32),
        scratch_shapes=[
            pltpu.VMEM((NCLS, INNER), jnp.float32),
            pltpu.VMEM((NCLS, INNER), jnp.float32),
            pltpu.VMEM((NCLS, INNER, BLK), jnp.float32),
        ],
        compiler_params=pltpu.CompilerParams(
            dimension_semantics=("arbitrary",)
        ),
        interpret=interpret,
    )(p6, c3, queues)


def kernel(fea, res, queues):
    res_flat = res.reshape(-1)
    fea_flat = fea.reshape(-1)
    pred16, countsp = _argmax_call()(res_flat)
    partials = _segsum_call()(fea_flat, pred16)
    p6 = (
        partials.reshape(BSZ, 4, NCLS, CHG, L)
        .transpose(2, 1, 3, 0, 4)
        .reshape(NCLS, INNER, BSZ * L)
    )
    c3 = countsp.reshape(NW, NCLS, L).transpose(1, 0, 2).reshape(NCLS, NW * L)
    loss = _loss_call(p6, c3, queues)[0, 0]
    return res, loss
